# baseline (device time: 9778 ns/iter reference)
import jax
import jax.numpy as jnp
from jax import lax
from jax.experimental import pallas as pl
from jax.experimental.pallas import tpu as pltpu


def kernel(x, pi):
    x = pltpu.with_memory_space_constraint(x, pltpu.MemorySpace.HBM)

    def body(x_ref, pi_ref, out_ref, send_sem, recv_sem, local_sem):
        my_x = lax.axis_index("x")
        my_y = lax.axis_index("y")
        my_z = lax.axis_index("z")
        dst_x = pi_ref[my_x]

        barrier_sem = pltpu.get_barrier_semaphore()
        pl.semaphore_signal(
            barrier_sem,
            inc=1,
            device_id=(1 - my_x, my_y, my_z),
            device_id_type=pl.DeviceIdType.MESH,
        )
        pl.semaphore_wait(barrier_sem, 1)

        @pl.when(dst_x == my_x)
        def _():
            copy = pltpu.make_async_copy(x_ref, out_ref, local_sem)
            copy.start()
            copy.wait()

        @pl.when(dst_x != my_x)
        def _():
            rdma = pltpu.make_async_remote_copy(
                src_ref=x_ref,
                dst_ref=out_ref,
                send_sem=send_sem,
                recv_sem=recv_sem,
                device_id=(dst_x, my_y, my_z),
                device_id_type=pl.DeviceIdType.MESH,
            )
            rdma.start()
            rdma.wait()

    return pl.pallas_call(
        body,
        out_shape=pltpu.MemorySpace.HBM(x.shape, jnp.float32),
        in_specs=[
            pl.BlockSpec(memory_space=pltpu.MemorySpace.HBM),
            pl.BlockSpec(memory_space=pltpu.SMEM),
        ],
        out_specs=pl.BlockSpec(memory_space=pltpu.MemorySpace.HBM),
        scratch_shapes=[
            pltpu.SemaphoreType.DMA,
            pltpu.SemaphoreType.DMA,
            pltpu.SemaphoreType.DMA,
        ],
        compiler_params=pltpu.CompilerParams(collective_id=0),
    )(x, pi)


# device time: 8417 ns/iter; 1.1617x vs baseline; 1.1617x over previous
import jax
import jax.numpy as jnp
from jax import lax
from jax.experimental import pallas as pl
from jax.experimental.pallas import tpu as pltpu


def kernel(x, pi):
    x = pltpu.with_memory_space_constraint(x, pltpu.MemorySpace.HBM)
    pi = pltpu.with_memory_space_constraint(pi, pltpu.MemorySpace.HBM)

    n_chunks = 2
    rows = x.shape[1] // n_chunks

    def body(x_ref, pi_ref, out_ref, x_vmem, pi_smem, sems, send_sems, recv_sems):
        my_x = lax.axis_index("x")
        my_y = lax.axis_index("y")
        my_z = lax.axis_index("z")

        copy_pi = pltpu.make_async_copy(pi_ref, pi_smem, sems.at[0])
        copy_pi.start()
        chunk_copies = []
        for c in range(n_chunks):
            sl = pl.ds(c * rows, rows)
            cp = pltpu.make_async_copy(
                x_ref.at[:, sl], x_vmem.at[:, sl], sems.at[1 + c]
            )
            cp.start()
            chunk_copies.append(cp)

        barrier_sem = pltpu.get_barrier_semaphore()
        pl.semaphore_signal(
            barrier_sem,
            inc=1,
            device_id=(1 - my_x, my_y, my_z),
            device_id_type=pl.DeviceIdType.MESH,
        )
        pl.semaphore_wait(barrier_sem, 1)

        copy_pi.wait()
        dst_x = pi_smem[my_x]

        @pl.when(dst_x == my_x)
        def _():
            for cp in chunk_copies:
                cp.wait()
            out_ref[...] = x_vmem[...]

        @pl.when(dst_x != my_x)
        def _():
            rdmas = []
            for c in range(n_chunks):
                sl = pl.ds(c * rows, rows)
                chunk_copies[c].wait()
                rdma = pltpu.make_async_remote_copy(
                    src_ref=x_vmem.at[:, sl],
                    dst_ref=out_ref.at[:, sl],
                    send_sem=send_sems.at[c],
                    recv_sem=recv_sems.at[c],
                    device_id=(dst_x, my_y, my_z),
                    device_id_type=pl.DeviceIdType.MESH,
                )
                rdma.start()
                rdmas.append(rdma)
            for rdma in rdmas:
                rdma.wait()

    return pl.pallas_call(
        body,
        out_shape=jax.ShapeDtypeStruct(x.shape, jnp.float32),
        in_specs=[
            pl.BlockSpec(memory_space=pltpu.MemorySpace.HBM),
            pl.BlockSpec(memory_space=pltpu.MemorySpace.HBM),
        ],
        out_specs=pl.BlockSpec(memory_space=pltpu.VMEM),
        scratch_shapes=[
            pltpu.VMEM(x.shape, jnp.float32),
            pltpu.SMEM(pi.shape, jnp.int32),
            pltpu.SemaphoreType.DMA((1 + n_chunks,)),
            pltpu.SemaphoreType.DMA((n_chunks,)),
            pltpu.SemaphoreType.DMA((n_chunks,)),
        ],
        compiler_params=pltpu.CompilerParams(collective_id=0),
    )(x, pi)


# device time: 8301 ns/iter; 1.1779x vs baseline; 1.0140x over previous
import jax
import jax.numpy as jnp
from jax import lax
from jax.experimental import pallas as pl
from jax.experimental.pallas import tpu as pltpu


def kernel(x, pi):
    x = pltpu.with_memory_space_constraint(x, pltpu.MemorySpace.HBM)
    pi = pltpu.with_memory_space_constraint(pi, pltpu.MemorySpace.HBM)

    def body(x_ref, pi_ref, out_ref, pi_smem, pi_sem, send_sem, recv_sem):
        my_x = lax.axis_index("x")
        my_y = lax.axis_index("y")
        my_z = lax.axis_index("z")

        copy_pi = pltpu.make_async_copy(pi_ref, pi_smem, pi_sem)
        copy_pi.start()

        barrier_sem = pltpu.get_barrier_semaphore()
        pl.semaphore_signal(
            barrier_sem,
            inc=1,
            device_id=(1 - my_x, my_y, my_z),
            device_id_type=pl.DeviceIdType.MESH,
        )
        pl.semaphore_wait(barrier_sem, 1)

        copy_pi.wait()
        dst_x = pi_smem[my_x]

        rdma = pltpu.make_async_remote_copy(
            src_ref=x_ref,
            dst_ref=out_ref,
            send_sem=send_sem,
            recv_sem=recv_sem,
            device_id=(dst_x, my_y, my_z),
            device_id_type=pl.DeviceIdType.MESH,
        )
        rdma.start()
        rdma.wait()

    return pl.pallas_call(
        body,
        out_shape=jax.ShapeDtypeStruct(x.shape, jnp.float32),
        in_specs=[
            pl.BlockSpec(memory_space=pltpu.MemorySpace.HBM),
            pl.BlockSpec(memory_space=pltpu.MemorySpace.HBM),
        ],
        out_specs=pl.BlockSpec(memory_space=pltpu.MemorySpace.HBM),
        scratch_shapes=[
            pltpu.SMEM(pi.shape, jnp.int32),
            pltpu.SemaphoreType.DMA,
            pltpu.SemaphoreType.DMA,
            pltpu.SemaphoreType.DMA,
        ],
        compiler_params=pltpu.CompilerParams(collective_id=0),
    )(x, pi)
